# baseline (speedup 1.0000x reference)
import jax
import jax.numpy as jnp
from jax.experimental import pallas as pl

_BS = 64


def _body(colT_ref, rowT_ref, o_ref):
    d, w = colT_ref.shape
    h = rowT_ref.shape[1]
    colT = colT_ref[...]
    rowT = rowT_ref[...]
    o_ref[0, :d] = jnp.broadcast_to(colT[:, None, :], (d, h, w))
    o_ref[0, d:] = jnp.broadcast_to(rowT[:, :, None], (d, h, w))


def kernel(row_embed, col_embed, bs):
    h, d = row_embed.shape
    w = col_embed.shape[0]
    colT = col_embed.T
    rowT = row_embed.T
    out = pl.pallas_call(
        _body,
        grid=(_BS,),
        in_specs=[
            pl.BlockSpec((d, w), lambda b: (0, 0)),
            pl.BlockSpec((d, h), lambda b: (0, 0)),
        ],
        out_specs=pl.BlockSpec((1, 2 * d, h, w), lambda b: (b, 0, 0, 0)),
        out_shape=jax.ShapeDtypeStruct((_BS, 2 * d, h, w), jnp.float32),
    )(colT, rowT)
    return out


# speedup vs baseline: 3.0060x; 3.0060x over previous
import jax
import jax.numpy as jnp
from jax.experimental import pallas as pl
from jax.experimental.pallas import tpu as pltpu

_BS = 64


def _body(colT_ref, rowT_ref, o_hbm, pos, sem):
    d, w = colT_ref.shape
    h = rowT_ref.shape[1]
    colT = colT_ref[...]
    for y in range(h):
        pos[0:d, y * w:(y + 1) * w] = colT
        pos[d:2 * d, y * w:(y + 1) * w] = jnp.broadcast_to(
            rowT_ref[:, y:y + 1], (d, w))
    copies = [pltpu.make_async_copy(pos, o_hbm.at[b], sem) for b in range(_BS)]
    for c in copies:
        c.start()
    for c in copies:
        c.wait()


def kernel(row_embed, col_embed, bs):
    h, d = row_embed.shape
    w = col_embed.shape[0]
    colT = col_embed.T
    rowT = row_embed.T
    out = pl.pallas_call(
        _body,
        in_specs=[
            pl.BlockSpec((d, w), lambda: (0, 0)),
            pl.BlockSpec((d, h), lambda: (0, 0)),
        ],
        out_specs=pl.BlockSpec(memory_space=pl.ANY),
        out_shape=jax.ShapeDtypeStruct((_BS, 2 * d, h * w), jnp.float32),
        scratch_shapes=[
            pltpu.VMEM((2 * d, h * w), jnp.float32),
            pltpu.SemaphoreType.DMA,
        ],
    )(colT, rowT)
    return out.reshape(_BS, 2 * d, h, w)
